# baseline (speedup 1.0000x reference)
import functools

import jax
import jax.numpy as jnp
from jax import lax
from jax.experimental import pallas as pl
from jax.experimental.pallas import tpu as pltpu
from jax.experimental.pallas import tpu_sc as plsc

NC, NS = 2, 16
NW = NC * NS


def _sc_gather(wids, cids, word_table, ct_flat, T, L, WD, CD):
    TW = T // NW
    CT = 128
    CHV = CT * L * CD
    NG = (CT * L) // 16
    assert TW % CT == 0 and (CT * L) % 16 == 0

    mesh = plsc.VectorSubcoreMesh(core_axis_name="c", subcore_axis_name="s")

    @functools.partial(
        pl.kernel,
        mesh=mesh,
        compiler_params=pltpu.CompilerParams(needs_layout_passes=False),
        out_type=(
            jax.ShapeDtypeStruct((T, WD), jnp.float32),
            jax.ShapeDtypeStruct((T * L * CD,), jnp.float32),
        ),
        scratch_types=[
            pltpu.VMEM((CT,), jnp.int32),
            pltpu.VMEM((CT * L,), jnp.int32),
            pltpu.VMEM((CT, WD), jnp.float32),
            pltpu.VMEM((CHV,), jnp.float32),
            pltpu.VMEM((ct_flat.shape[0],), jnp.float32),
            pltpu.SemaphoreType.DMA,
        ],
    )
    def sck(wids_h, cids_h, wt_h, ct_h, wout_h, ceout_h,
            widv, cidv, wrows, cerows, ctv, s1):
        wid = lax.axis_index("s") * NC + lax.axis_index("c")
        pltpu.sync_copy(ct_h, ctv)
        iota = lax.iota(jnp.int32, 16)
        sidx = [iota * CD + c for c in range(CD)]

        def body(c, carry):
            base = pl.multiple_of(wid * TW + c * CT, CT)
            pltpu.sync_copy(wids_h.at[pl.ds(base, CT)], widv)
            pltpu.sync_copy(cids_h.at[pl.ds(pl.multiple_of(base * L, 8),
                                            CT * L)], cidv)
            cw = pltpu.async_copy(wt_h.at[widv], wrows, s1)

            def gather_group(g, carry2):
                cidvec = cidv[pl.ds(pl.multiple_of(g * 16, 16), 16)]
                b16 = cidvec * CD
                goff = g * (16 * CD)
                for ch in range(CD):
                    v = plsc.load_gather(ctv, [b16 + ch])
                    plsc.store_scatter(cerows, [sidx[ch] + goff], v)
                return carry2

            lax.fori_loop(0, NG, gather_group, 0)
            cw.wait()
            pltpu.sync_copy(wrows, wout_h.at[pl.ds(base, CT)])
            pltpu.sync_copy(cerows, ceout_h.at[pl.ds(
                pl.multiple_of(base * (L * CD), 8), CHV)])
            return carry

        lax.fori_loop(0, TW // CT, body, 0)

    return sck(wids, cids, word_table, ct_flat)


def _tc_conv_assemble(ce2, word_raw, wbig, wids_col, bias_row, T, L, WD, CV):
    TB = 1024
    assert T % TB == 0
    KD = ce2.shape[1]
    ND = wbig.shape[1]

    def tck(ce_ref, w_ref, wb_ref, ids_ref, b_ref, out_ref):
        y = jnp.dot(ce_ref[...].astype(jnp.bfloat16), wb_ref[...],
                    preferred_element_type=jnp.float32)
        y = y + b_ref[...]
        m = y[:, 0:CV]
        for p in range(1, L):
            m = jnp.maximum(m, y[:, p * CV:(p + 1) * CV])
        mask = (ids_ref[...] != 0).astype(jnp.float32)
        out_ref[:, :WD] = w_ref[...] * mask
        out_ref[:, WD:] = m

    return pl.pallas_call(
        tck,
        grid=(T // TB,),
        in_specs=[
            pl.BlockSpec((TB, KD), lambda i: (i, 0)),
            pl.BlockSpec((TB, WD), lambda i: (i, 0)),
            pl.BlockSpec((KD, ND), lambda i: (0, 0)),
            pl.BlockSpec((TB, 1), lambda i: (i, 0)),
            pl.BlockSpec((1, ND), lambda i: (0, 0)),
        ],
        out_specs=pl.BlockSpec((TB, WD + CV), lambda i: (i, 0)),
        out_shape=jax.ShapeDtypeStruct((T, WD + CV), jnp.float32),
    )(ce2, word_raw, wbig, wids_col, bias_row)


def kernel(b_word_ids, b_char_ids, word_table, char_table, conv_w, conv_b):
    B, S = b_word_ids.shape
    L = b_char_ids.shape[2]
    WD = word_table.shape[1]
    CD = char_table.shape[1]
    CV = conv_w.shape[0]
    T = B * S

    wids = b_word_ids.reshape(T)
    cids = b_char_ids.reshape(T * L)
    ct0 = char_table.at[0].set(0.0)

    word_raw, ce = _sc_gather(wids, cids, word_table, ct0.reshape(-1),
                              T, L, WD, CD)

    W4 = jnp.zeros((L, CD, L, CV), jnp.float32)
    for p in range(L):
        for k in range(5):
            l = p + k - 2
            if 0 <= l < L:
                W4 = W4.at[l, :, p, :].set(conv_w[:, :, k].T)
    wbig = W4.reshape(L * CD, L * CV).astype(jnp.bfloat16)
    bias_row = jnp.tile(conv_b, L)[None, :]

    ce2 = ce.reshape(T, L * CD)
    out = _tc_conv_assemble(ce2, word_raw, wbig, wids.reshape(T, 1),
                            bias_row, T, L, WD, CV)
    return out.reshape(B, S, WD + CV)


# speedup vs baseline: 1.5781x; 1.5781x over previous
import functools

import jax
import jax.numpy as jnp
from jax import lax
from jax.experimental import pallas as pl
from jax.experimental.pallas import tpu as pltpu
from jax.experimental.pallas import tpu_sc as plsc

NC, NS = 2, 16
NW = NC * NS


def _sc_gather(wids, cids, word_table, char_table, T, L, WD, CD):
    TW = T // NW
    CT = 128
    NSUB = (CT * L) // 128
    assert TW % CT == 0 and (CT * L) % 128 == 0

    mesh = plsc.VectorSubcoreMesh(core_axis_name="c", subcore_axis_name="s")

    @functools.partial(
        pl.kernel,
        mesh=mesh,
        compiler_params=pltpu.CompilerParams(use_tc_tiling_on_sc=False),
        out_type=(
            jax.ShapeDtypeStruct((T, WD), jnp.float32),
            jax.ShapeDtypeStruct((T * L, CD), jnp.float32),
        ),
        scratch_types=[
            pltpu.VMEM((CT,), jnp.int32),
            pltpu.VMEM((CT * L,), jnp.int32),
            pltpu.VMEM((CT, WD), jnp.float32),
            pltpu.VMEM((CT * L, CD), jnp.float32),
            pltpu.SemaphoreType.DMA,
            pltpu.SemaphoreType.DMA,
        ],
    )
    def sck(wids_h, cids_h, wt_h, ct_h, wout_h, ceout_h,
            widv, cidv, wrows, cerows, s1, s2):
        wid = lax.axis_index("s") * NC + lax.axis_index("c")

        def body(c, carry):
            base = wid * TW + c * CT
            pltpu.sync_copy(wids_h.at[pl.ds(base, CT)], widv)
            pltpu.sync_copy(cids_h.at[pl.ds(base * L, CT * L)], cidv)
            cw = pltpu.async_copy(wt_h.at[widv], wrows, s1)
            chs = []
            for j in range(NSUB):
                chs.append(pltpu.async_copy(
                    ct_h.at[cidv.at[pl.ds(j * 128, 128)]],
                    cerows.at[pl.ds(j * 128, 128)], s2))
            cw.wait()
            for h in chs:
                h.wait()
            pltpu.sync_copy(wrows, wout_h.at[pl.ds(base, CT)])
            pltpu.sync_copy(cerows, ceout_h.at[pl.ds(base * L, CT * L)])
            return carry

        lax.fori_loop(0, TW // CT, body, 0)

    return sck(wids, cids, word_table, char_table)


def _tc_conv_assemble(ce2, word_raw, wbig, wids_col, bias_row,
                      B, S, T, L, WD, CV):
    TB = 1024
    assert T % TB == 0 and B % TB == 0
    BLK_PER_S = B // TB
    KD = ce2.shape[1]
    ND = wbig.shape[1]

    def tck(ce_ref, w_ref, wb_ref, ids_ref, b_ref, out_ref):
        y = jnp.dot(ce_ref[...].astype(jnp.bfloat16), wb_ref[...],
                    preferred_element_type=jnp.float32)
        y = y + b_ref[...]
        m = y[:, 0:CV]
        for p in range(1, L):
            m = jnp.maximum(m, y[:, p * CV:(p + 1) * CV])
        mask = (ids_ref[...] != 0).astype(jnp.float32)
        wm = w_ref[...] * mask
        out_ref[0, :WD, :] = wm.T
        out_ref[0, WD:, :] = m.T

    out3 = pl.pallas_call(
        tck,
        grid=(T // TB,),
        in_specs=[
            pl.BlockSpec((TB, KD), lambda i: (i, 0)),
            pl.BlockSpec((TB, WD), lambda i: (i, 0)),
            pl.BlockSpec((KD, ND), lambda i: (0, 0)),
            pl.BlockSpec((TB, 1), lambda i: (i, 0)),
            pl.BlockSpec((1, ND), lambda i: (0, 0)),
        ],
        out_specs=pl.BlockSpec(
            (1, WD + CV, TB),
            lambda i: (i // BLK_PER_S, 0, i % BLK_PER_S)),
        out_shape=jax.ShapeDtypeStruct((S, WD + CV, B), jnp.float32),
    )(ce2, word_raw, wbig, wids_col, bias_row)
    return jnp.transpose(out3, (2, 0, 1))


def kernel(b_word_ids, b_char_ids, word_table, char_table, conv_w, conv_b):
    B, S = b_word_ids.shape
    L = b_char_ids.shape[2]
    WD = word_table.shape[1]
    CD = char_table.shape[1]
    CV = conv_w.shape[0]
    T = B * S

    wids = b_word_ids.T.reshape(T)
    cids = jnp.transpose(b_char_ids, (1, 0, 2)).reshape(T * L)
    ct0 = char_table.at[0].set(0.0)

    word_raw, ce = _sc_gather(wids, cids, word_table, ct0, T, L, WD, CD)

    W4 = jnp.zeros((L, CD, L, CV), jnp.float32)
    for p in range(L):
        for k in range(5):
            l = p + k - 2
            if 0 <= l < L:
                W4 = W4.at[l, :, p, :].set(conv_w[:, :, k].T)
    wbig = W4.reshape(L * CD, L * CV).astype(jnp.bfloat16)
    bias_row = jnp.tile(conv_b, L)[None, :]

    ce2 = ce.reshape(T, L * CD)
    return _tc_conv_assemble(ce2, word_raw, wbig, wids.reshape(T, 1),
                             bias_row, B, S, T, L, WD, CV)


# speedup vs baseline: 1.8444x; 1.1687x over previous
import functools

import jax
import jax.numpy as jnp
from jax import lax
from jax.experimental import pallas as pl
from jax.experimental.pallas import tpu as pltpu
from jax.experimental.pallas import tpu_sc as plsc

NC, NS = 2, 16
NW = NC * NS


def _sc_gather(wids, cids, word_table, char_table, T, L, WD, CD):
    TW = T // NW
    CT = 128
    NSUB = (CT * L) // 128
    assert TW % CT == 0 and (CT * L) % 128 == 0

    mesh = plsc.VectorSubcoreMesh(core_axis_name="c", subcore_axis_name="s")

    @functools.partial(
        pl.kernel,
        mesh=mesh,
        compiler_params=pltpu.CompilerParams(use_tc_tiling_on_sc=False),
        out_type=(
            jax.ShapeDtypeStruct((T, WD), jnp.float32),
            jax.ShapeDtypeStruct((T * L, CD), jnp.float32),
        ),
        scratch_types=[
            pltpu.VMEM((CT,), jnp.int32),
            pltpu.VMEM((CT * L,), jnp.int32),
            pltpu.VMEM((CT, WD), jnp.float32),
            pltpu.VMEM((CT * L, CD), jnp.float32),
            pltpu.SemaphoreType.DMA,
            pltpu.SemaphoreType.DMA,
        ],
    )
    def sck(wids_h, cids_h, wt_h, ct_h, wout_h, ceout_h,
            widv, cidv, wrows, cerows, s1, s2):
        wid = lax.axis_index("s") * NC + lax.axis_index("c")

        def body(c, carry):
            base = wid * TW + c * CT
            pltpu.sync_copy(wids_h.at[pl.ds(base, CT)], widv)
            pltpu.sync_copy(cids_h.at[pl.ds(base * L, CT * L)], cidv)
            cw = pltpu.async_copy(wt_h.at[widv], wrows, s1)
            chs = []
            for j in range(NSUB):
                chs.append(pltpu.async_copy(
                    ct_h.at[cidv.at[pl.ds(j * 128, 128)]],
                    cerows.at[pl.ds(j * 128, 128)], s2))
            cw.wait()
            for h in chs:
                h.wait()
            pltpu.sync_copy(wrows, wout_h.at[pl.ds(base, CT)])
            pltpu.sync_copy(cerows, ceout_h.at[pl.ds(base * L, CT * L)])
            return carry

        lax.fori_loop(0, TW // CT, body, 0)

    return sck(wids, cids, word_table, char_table)


def _tc_conv_assemble(ce2, word_raw, wbig, wids_col, bias_row,
                      B, S, T, L, WD, CV):
    TB = 2048
    assert T % TB == 0 and B % TB == 0
    BLK_PER_S = B // TB
    KD = ce2.shape[1]
    ND = wbig.shape[1]

    def tck(ce_ref, w_ref, wb_ref, ids_ref, b_ref, out_ref):
        y = jnp.dot(ce_ref[...].astype(jnp.bfloat16), wb_ref[...],
                    preferred_element_type=jnp.float32)
        m128 = jnp.maximum(jnp.maximum(y[:, 0:128], y[:, 128:256]),
                           y[:, 256:384])
        m64 = jnp.maximum(m128[:, 0:64], m128[:, 64:128])
        m = jnp.maximum(m64[:, 0:32], m64[:, 32:64]) + b_ref[...]
        mask = (ids_ref[0] != 0).astype(jnp.float32)
        out_ref[0, :WD, :] = w_ref[...].T * mask
        out_ref[0, WD:, :] = m.T

    out3 = pl.pallas_call(
        tck,
        grid=(T // TB,),
        in_specs=[
            pl.BlockSpec((TB, KD), lambda i: (i, 0)),
            pl.BlockSpec((TB, WD), lambda i: (i, 0)),
            pl.BlockSpec((KD, ND), lambda i: (0, 0)),
            pl.BlockSpec((1, 1, TB), lambda i: (i, 0, 0)),
            pl.BlockSpec((1, CV), lambda i: (0, 0)),
        ],
        out_specs=pl.BlockSpec(
            (1, WD + CV, TB),
            lambda i: (i // BLK_PER_S, 0, i % BLK_PER_S)),
        out_shape=jax.ShapeDtypeStruct((S, WD + CV, B), jnp.float32),
    )(ce2, word_raw, wbig, wids_col, bias_row)
    return jnp.transpose(out3, (2, 0, 1))


def kernel(b_word_ids, b_char_ids, word_table, char_table, conv_w, conv_b):
    B, S = b_word_ids.shape
    L = b_char_ids.shape[2]
    WD = word_table.shape[1]
    CD = char_table.shape[1]
    CV = conv_w.shape[0]
    T = B * S

    wids = b_word_ids.T.reshape(T)
    cids = jnp.transpose(b_char_ids, (1, 0, 2)).reshape(T * L)
    ct0 = char_table.at[0].set(0.0)

    word_raw, ce = _sc_gather(wids, cids, word_table, ct0, T, L, WD, CD)

    W4 = jnp.zeros((L, CD, L, CV), jnp.float32)
    for p in range(L):
        for k in range(5):
            l = p + k - 2
            if 0 <= l < L:
                W4 = W4.at[l, :, p, :].set(conv_w[:, :, k].T)
    wbig = W4.reshape(L * CD, L * CV).astype(jnp.bfloat16)
    bias_row = conv_b[None, :]

    ce2 = ce.reshape(T, L * CD)
    return _tc_conv_assemble(ce2, word_raw, wbig, wids.reshape(T // 2048, 1, 2048),
                             bias_row, B, S, T, L, WD, CV)


# speedup vs baseline: 1.8545x; 1.0055x over previous
import functools

import jax
import jax.numpy as jnp
from jax import lax
from jax.experimental import pallas as pl
from jax.experimental.pallas import tpu as pltpu
from jax.experimental.pallas import tpu_sc as plsc

NC, NS = 2, 16
NW = NC * NS


def _sc_gather(wids, cids, word_table, char_table, T, L, WD, CD):
    TW = T // NW
    CT = 128
    NSUB = (CT * L) // 128
    assert TW % CT == 0 and (CT * L) % 128 == 0

    mesh = plsc.VectorSubcoreMesh(core_axis_name="c", subcore_axis_name="s")

    @functools.partial(
        pl.kernel,
        mesh=mesh,
        compiler_params=pltpu.CompilerParams(use_tc_tiling_on_sc=False),
        out_type=(
            jax.ShapeDtypeStruct((T, WD), jnp.float32),
            jax.ShapeDtypeStruct((T * L, CD), jnp.float32),
        ),
        scratch_types=[
            pltpu.VMEM((2, CT), jnp.int32),
            pltpu.VMEM((2, CT * L), jnp.int32),
            pltpu.VMEM((2, CT, WD), jnp.float32),
            pltpu.VMEM((2, CT * L, CD), jnp.float32),
            pltpu.SemaphoreType.DMA,
            pltpu.SemaphoreType.DMA,
            pltpu.SemaphoreType.DMA,
            pltpu.SemaphoreType.DMA,
        ],
    )
    def sck(wids_h, cids_h, wt_h, ct_h, wout_h, ceout_h,
            widv, cidv, wrows, cerows, sw0, sw1, sc0, sc1):
        wid = lax.axis_index("s") * NC + lax.axis_index("c")
        sw = (sw0, sw1)
        sc = (sc0, sc1)
        NCH = TW // CT

        def stage_fire(c, p):
            base = c * CT + wid * TW
            pltpu.sync_copy(wids_h.at[pl.ds(base, CT)], widv.at[p])
            pltpu.sync_copy(cids_h.at[pl.ds(base * L, CT * L)], cidv.at[p])
            pltpu.async_copy(wt_h.at[widv.at[p]], wrows.at[p], sw[p])
            for j in range(NSUB):
                pltpu.async_copy(
                    ct_h.at[cidv.at[p].at[pl.ds(j * 128, 128)]],
                    cerows.at[p].at[pl.ds(j * 128, 128)], sc[p])

        def drain_write(c, p):
            base = c * CT + wid * TW
            pltpu.make_async_copy(wt_h.at[widv.at[p]], wrows.at[p],
                                  sw[p]).wait()
            for j in range(NSUB):
                pltpu.make_async_copy(
                    ct_h.at[cidv.at[p].at[pl.ds(j * 128, 128)]],
                    cerows.at[p].at[pl.ds(j * 128, 128)], sc[p]).wait()
            pltpu.sync_copy(wrows.at[p], wout_h.at[pl.ds(base, CT)])
            pltpu.sync_copy(cerows.at[p],
                            ceout_h.at[pl.ds(base * L, CT * L)])

        stage_fire(0, 0)

        def body(g, carry):
            c = g * 2
            stage_fire(c + 1, 1)
            drain_write(c, 0)
            stage_fire(c + 2, 0)
            drain_write(c + 1, 1)
            return carry

        lax.fori_loop(0, NCH // 2 - 1, body, 0)
        c_last = NCH - 2
        stage_fire(c_last + 1, 1)
        drain_write(c_last, 0)
        drain_write(c_last + 1, 1)

    return sck(wids, cids, word_table, char_table)


def _tc_conv_assemble(ce2, word_raw, wbig, wids_col, bias_row,
                      B, S, T, L, WD, CV):
    TB = 2048
    assert T % TB == 0 and B % TB == 0
    BLK_PER_S = B // TB
    KD = ce2.shape[1]
    ND = wbig.shape[1]

    def tck(ce_ref, w_ref, wb_ref, ids_ref, b_ref, out_ref):
        y = jnp.dot(ce_ref[...].astype(jnp.bfloat16), wb_ref[...],
                    preferred_element_type=jnp.float32)
        m128 = jnp.maximum(jnp.maximum(y[:, 0:128], y[:, 128:256]),
                           y[:, 256:384])
        m64 = jnp.maximum(m128[:, 0:64], m128[:, 64:128])
        m = jnp.maximum(m64[:, 0:32], m64[:, 32:64]) + b_ref[...]
        mask = (ids_ref[0] != 0).astype(jnp.float32)
        out_ref[0, :WD, :] = w_ref[...].T * mask
        out_ref[0, WD:, :] = m.T

    out3 = pl.pallas_call(
        tck,
        grid=(T // TB,),
        in_specs=[
            pl.BlockSpec((TB, KD), lambda i: (i, 0)),
            pl.BlockSpec((TB, WD), lambda i: (i, 0)),
            pl.BlockSpec((KD, ND), lambda i: (0, 0)),
            pl.BlockSpec((1, 1, TB), lambda i: (i, 0, 0)),
            pl.BlockSpec((1, CV), lambda i: (0, 0)),
        ],
        out_specs=pl.BlockSpec(
            (1, WD + CV, TB),
            lambda i: (i // BLK_PER_S, 0, i % BLK_PER_S)),
        out_shape=jax.ShapeDtypeStruct((S, WD + CV, B), jnp.float32),
    )(ce2, word_raw, wbig, wids_col, bias_row)
    return jnp.transpose(out3, (2, 0, 1))


def kernel(b_word_ids, b_char_ids, word_table, char_table, conv_w, conv_b):
    B, S = b_word_ids.shape
    L = b_char_ids.shape[2]
    WD = word_table.shape[1]
    CD = char_table.shape[1]
    CV = conv_w.shape[0]
    T = B * S

    wids = b_word_ids.T.reshape(T)
    cids = jnp.transpose(b_char_ids, (1, 0, 2)).reshape(T * L)
    ct0 = char_table.at[0].set(0.0)

    word_raw, ce = _sc_gather(wids, cids, word_table, ct0, T, L, WD, CD)

    W4 = jnp.zeros((L, CD, L, CV), jnp.float32)
    for p in range(L):
        for k in range(5):
            l = p + k - 2
            if 0 <= l < L:
                W4 = W4.at[l, :, p, :].set(conv_w[:, :, k].T)
    wbig = W4.reshape(L * CD, L * CV).astype(jnp.bfloat16)
    bias_row = conv_b[None, :]

    ce2 = ce.reshape(T, L * CD)
    return _tc_conv_assemble(ce2, word_raw, wbig, wids.reshape(T // 2048, 1, 2048),
                             bias_row, B, S, T, L, WD, CV)


# speedup vs baseline: 2.5469x; 1.3734x over previous
import functools

import jax
import jax.numpy as jnp
from jax import lax
from jax.experimental import pallas as pl
from jax.experimental.pallas import tpu as pltpu
from jax.experimental.pallas import tpu_sc as plsc

NC, NS = 2, 16
NW = NC * NS


def _sc_gather(wids, cids, word_table, char_table, T, L, WD, CD):
    TW = T // NW
    CT = 128
    NSUB = (CT * L) // 128
    assert TW % CT == 0 and (CT * L) % 128 == 0

    mesh = plsc.VectorSubcoreMesh(core_axis_name="c", subcore_axis_name="s")

    @functools.partial(
        pl.kernel,
        mesh=mesh,
        compiler_params=pltpu.CompilerParams(use_tc_tiling_on_sc=False,
                                             needs_layout_passes=False),
        out_type=(
            jax.ShapeDtypeStruct((T, WD), jnp.float32),
            jax.ShapeDtypeStruct((L * CD // 8, T // 128, 8, 128),
                                 jnp.float32),
        ),
        scratch_types=[
            pltpu.VMEM((2, CT), jnp.int32),
            pltpu.VMEM((2, CT * L), jnp.int32),
            pltpu.VMEM((2, CT, WD), jnp.float32),
            pltpu.VMEM((2, CT * L, CD), jnp.float32),
            pltpu.VMEM((L * CD, CT + 1), jnp.float32),
            pltpu.SemaphoreType.DMA,
            pltpu.SemaphoreType.DMA,
            pltpu.SemaphoreType.DMA,
            pltpu.SemaphoreType.DMA,
            pltpu.SemaphoreType.DMA,
        ],
    )
    def sck(wids_h, cids_h, wt_h, ct_h, wout_h, ceout_h,
            widv, cidv, wrows, cerows, cet, sw0, sw1, sc0, sc1, st):
        wid = lax.axis_index("s") * NC + lax.axis_index("c")
        sw = (sw0, sw1)
        sc = (sc0, sc1)
        NCH = TW // CT
        iota = lax.iota(jnp.int32, 16)
        sidx = [l * CD + iota for l in range(L)]

        def stage_fire(c, p):
            base = c * CT + wid * TW
            pltpu.sync_copy(wids_h.at[pl.ds(base, CT)], widv.at[p])
            pltpu.sync_copy(cids_h.at[pl.ds(base * L, CT * L)], cidv.at[p])
            pltpu.async_copy(wt_h.at[widv.at[p]], wrows.at[p], sw[p])
            cer2 = cerows.at[p]
            for j in range(NSUB):
                pltpu.async_copy(
                    ct_h.at[cidv.at[p].at[pl.ds(j * 128, 128)]],
                    cer2.at[pl.ds(j * 128, 128)], sc[p])

        def drain_write(c, p):
            base = c * CT + wid * TW
            cg = base // CT
            cer2 = cerows.at[p]
            pltpu.make_async_copy(wt_h.at[widv.at[p]], wrows.at[p],
                                  sw[p]).wait()
            for j in range(NSUB):
                pltpu.make_async_copy(
                    ct_h.at[cidv.at[p].at[pl.ds(j * 128, 128)]],
                    cer2.at[pl.ds(j * 128, 128)], sc[p]).wait()

            def trans_tok(t, carry2):
                for l in range(L):
                    v = cer2[t * L + l]
                    plsc.store_scatter(cet, [sidx[l], jnp.full((16,), t,
                                                              jnp.int32)], v)
                return carry2

            lax.fori_loop(0, CT, trans_tok, 0, unroll=4)

            pltpu.sync_copy(wrows.at[p], wout_h.at[pl.ds(base, CT)])
            for r in range(L * CD // 8):
                pltpu.async_copy(
                    cet.at[pl.ds(r * 8, 8), pl.ds(0, CT)],
                    ceout_h.at[r].at[cg], st)
            for r in range(L * CD // 8):
                pltpu.make_async_copy(
                    cet.at[pl.ds(r * 8, 8), pl.ds(0, CT)],
                    ceout_h.at[r].at[cg], st).wait()

        stage_fire(0, 0)

        def body(g, carry):
            c = g * 2
            stage_fire(c + 1, 1)
            drain_write(c, 0)
            stage_fire(c + 2, 0)
            drain_write(c + 1, 1)
            return carry

        lax.fori_loop(0, NCH // 2 - 1, body, 0)
        c_last = NCH - 2
        stage_fire(c_last + 1, 1)
        drain_write(c_last, 0)
        drain_write(c_last + 1, 1)

    return sck(wids, cids, word_table, char_table)


def _tc_conv_assemble(ce2, word_raw, wbig, wids_col, bias_row,
                      B, S, T, L, WD, CV):
    TB = 2048
    assert T % TB == 0 and B % TB == 0
    BLK_PER_S = B // TB
    KD = ce2.shape[0]
    ND = wbig.shape[0]

    def tck(ce_ref, w_ref, wb_ref, ids_ref, b_ref, out_ref):
        y = jnp.dot(wb_ref[...], ce_ref[...].astype(jnp.bfloat16),
                    preferred_element_type=jnp.float32)
        m128 = jnp.maximum(jnp.maximum(y[0:128, :], y[128:256, :]),
                           y[256:384, :])
        m64 = jnp.maximum(m128[0:64, :], m128[64:128, :])
        m = jnp.maximum(m64[0:32, :], m64[32:64, :]) + b_ref[...]
        mask = (ids_ref[0] != 0).astype(jnp.float32)
        out_ref[0, :WD, :] = w_ref[...].T * mask
        out_ref[0, WD:, :] = m

    out3 = pl.pallas_call(
        tck,
        grid=(T // TB,),
        in_specs=[
            pl.BlockSpec((KD, TB), lambda i: (0, i)),
            pl.BlockSpec((TB, WD), lambda i: (i, 0)),
            pl.BlockSpec((ND, KD), lambda i: (0, 0)),
            pl.BlockSpec((1, 1, TB), lambda i: (i, 0, 0)),
            pl.BlockSpec((CV, 1), lambda i: (0, 0)),
        ],
        out_specs=pl.BlockSpec(
            (1, WD + CV, TB),
            lambda i: (i // BLK_PER_S, 0, i % BLK_PER_S)),
        out_shape=jax.ShapeDtypeStruct((S, WD + CV, B), jnp.float32),
    )(ce2, word_raw, wbig, wids_col, bias_row)
    return jnp.transpose(out3, (2, 0, 1))


def kernel(b_word_ids, b_char_ids, word_table, char_table, conv_w, conv_b):
    B, S = b_word_ids.shape
    L = b_char_ids.shape[2]
    WD = word_table.shape[1]
    CD = char_table.shape[1]
    CV = conv_w.shape[0]
    T = B * S

    wids = b_word_ids.T.reshape(T)
    cids = jnp.transpose(b_char_ids, (1, 0, 2)).reshape(T * L)
    ct0 = char_table.at[0].set(0.0)

    word_raw, ce = _sc_gather(wids, cids, word_table, ct0, T, L, WD, CD)

    W4 = jnp.zeros((L, CD, L, CV), jnp.float32)
    for p in range(L):
        for k in range(5):
            l = p + k - 2
            if 0 <= l < L:
                W4 = W4.at[l, :, p, :].set(conv_w[:, :, k].T)
    wbig = W4.reshape(L * CD, L * CV).T.astype(jnp.bfloat16)
    bias_row = conv_b[:, None]

    ce2 = jnp.transpose(ce, (0, 2, 1, 3)).reshape(L * CD, T)
    return _tc_conv_assemble(ce2, word_raw, wbig, wids.reshape(T // 2048, 1, 2048),
                             bias_row, B, S, T, L, WD, CV)


# speedup vs baseline: 3.0268x; 1.1884x over previous
import functools

import jax
import jax.numpy as jnp
from jax import lax
from jax.experimental import pallas as pl
from jax.experimental.pallas import tpu as pltpu
from jax.experimental.pallas import tpu_sc as plsc

NC, NS = 2, 16
NW = NC * NS


def _sc_gather(wids, cids, word_table, char_table, T, L, WD, CD):
    TW = T // NW
    CT = 128
    NSUB = (CT * L) // 128
    assert TW % CT == 0 and (CT * L) % 128 == 0

    mesh = plsc.VectorSubcoreMesh(core_axis_name="c", subcore_axis_name="s")

    @functools.partial(
        pl.kernel,
        mesh=mesh,
        compiler_params=pltpu.CompilerParams(use_tc_tiling_on_sc=False,
                                             needs_layout_passes=False),
        out_type=(
            jax.ShapeDtypeStruct((T, WD), jnp.float32),
            jax.ShapeDtypeStruct((L * CD // 8, T // 128, 8, 128),
                                 jnp.float32),
        ),
        scratch_types=[
            pltpu.VMEM((2, CT), jnp.int32),
            pltpu.VMEM((2, L, CT), jnp.int32),
            pltpu.VMEM((2, CT, WD), jnp.float32),
            pltpu.VMEM((2, CT * L, CD), jnp.float32),
            pltpu.VMEM((L * CD, CT + 1), jnp.float32),
            pltpu.SemaphoreType.DMA,
            pltpu.SemaphoreType.DMA,
            pltpu.SemaphoreType.DMA,
            pltpu.SemaphoreType.DMA,
            pltpu.SemaphoreType.DMA,
        ],
    )
    def sck(wids_h, cids_h, wt_h, ct_h, wout_h, ceout_h,
            widv, cidv, wrows, cerows, cet, sw0, sw1, sc0, sc1, st):
        wid = lax.axis_index("s") * NC + lax.axis_index("c")
        sw = (sw0, sw1)
        sc = (sc0, sc1)
        NCH = TW // CT
        iota = lax.iota(jnp.int32, 16)
        sidx = [l * CD + iota for l in range(L)]

        def stage_fire(c, p):
            base = c * CT + wid * TW
            pltpu.sync_copy(wids_h.at[pl.ds(base, CT)], widv.at[p])
            pltpu.sync_copy(cids_h.at[:, pl.ds(base, CT)], cidv.at[p])
            pltpu.async_copy(wt_h.at[widv.at[p]], wrows.at[p], sw[p])
            cer2 = cerows.at[p]
            for j in range(NSUB):
                pltpu.async_copy(
                    ct_h.at[cidv.at[p].at[j]],
                    cer2.at[pl.ds(j * 128, 128)], sc[p])

        def drain_write(c, p):
            base = c * CT + wid * TW
            cg = base // CT
            cer2 = cerows.at[p]
            pltpu.make_async_copy(wt_h.at[widv.at[p]], wrows.at[p],
                                  sw[p]).wait()
            for j in range(NSUB):
                pltpu.make_async_copy(
                    ct_h.at[cidv.at[p].at[j]],
                    cer2.at[pl.ds(j * 128, 128)], sc[p]).wait()

            def trans_tok(t, carry2):
                tv = jnp.full((16,), t, jnp.int32)
                for l in range(L):
                    v = cer2[l * CT + t]
                    plsc.store_scatter(cet, [sidx[l], tv], v)
                return carry2

            lax.fori_loop(0, CT, trans_tok, 0, unroll=4)

            pltpu.sync_copy(wrows.at[p], wout_h.at[pl.ds(base, CT)])
            for r in range(L * CD // 8):
                pltpu.async_copy(
                    cet.at[pl.ds(r * 8, 8), pl.ds(0, CT)],
                    ceout_h.at[r].at[cg], st)
            for r in range(L * CD // 8):
                pltpu.make_async_copy(
                    cet.at[pl.ds(r * 8, 8), pl.ds(0, CT)],
                    ceout_h.at[r].at[cg], st).wait()

        stage_fire(0, 0)

        def body(g, carry):
            c = g * 2
            stage_fire(c + 1, 1)
            drain_write(c, 0)
            stage_fire(c + 2, 0)
            drain_write(c + 1, 1)
            return carry

        lax.fori_loop(0, NCH // 2 - 1, body, 0)
        c_last = NCH - 2
        stage_fire(c_last + 1, 1)
        drain_write(c_last, 0)
        drain_write(c_last + 1, 1)

    return sck(wids, cids, word_table, char_table)


def _tc_conv_assemble(ce2, word_raw, wbig, wids_col, bias_row,
                      B, S, T, L, WD, CV):
    TB = 2048
    assert T % TB == 0 and B % TB == 0
    BLK_PER_S = B // TB
    KD = ce2.shape[0]
    ND = wbig.shape[0]

    def tck(ce_ref, w_ref, wb_ref, ids_ref, b_ref, out_ref):
        y = jnp.dot(wb_ref[...], ce_ref[...].astype(jnp.bfloat16),
                    preferred_element_type=jnp.float32)
        m128 = jnp.maximum(jnp.maximum(y[0:128, :], y[128:256, :]),
                           y[256:384, :])
        m64 = jnp.maximum(m128[0:64, :], m128[64:128, :])
        m = jnp.maximum(m64[0:32, :], m64[32:64, :]) + b_ref[...]
        mask = (ids_ref[0] != 0).astype(jnp.float32)
        out_ref[0, :WD, :] = w_ref[...].T * mask
        out_ref[0, WD:, :] = m

    out3 = pl.pallas_call(
        tck,
        grid=(T // TB,),
        in_specs=[
            pl.BlockSpec((KD, TB), lambda i: (0, i)),
            pl.BlockSpec((TB, WD), lambda i: (i, 0)),
            pl.BlockSpec((ND, KD), lambda i: (0, 0)),
            pl.BlockSpec((1, 1, TB), lambda i: (i, 0, 0)),
            pl.BlockSpec((CV, 1), lambda i: (0, 0)),
        ],
        out_specs=pl.BlockSpec(
            (1, WD + CV, TB),
            lambda i: (i // BLK_PER_S, 0, i % BLK_PER_S)),
        out_shape=jax.ShapeDtypeStruct((S, WD + CV, B), jnp.float32),
    )(ce2, word_raw, wbig, wids_col, bias_row)
    return jnp.transpose(out3, (2, 0, 1))


def kernel(b_word_ids, b_char_ids, word_table, char_table, conv_w, conv_b):
    B, S = b_word_ids.shape
    L = b_char_ids.shape[2]
    WD = word_table.shape[1]
    CD = char_table.shape[1]
    CV = conv_w.shape[0]
    T = B * S

    wids = b_word_ids.T.reshape(T)
    cids = jnp.transpose(b_char_ids, (2, 1, 0)).reshape(L, T)
    ct0 = char_table.at[0].set(0.0)

    word_raw, ce = _sc_gather(wids, cids, word_table, ct0, T, L, WD, CD)

    W4 = jnp.zeros((L, CD, L, CV), jnp.float32)
    for p in range(L):
        for k in range(5):
            l = p + k - 2
            if 0 <= l < L:
                W4 = W4.at[l, :, p, :].set(conv_w[:, :, k].T)
    wbig = W4.reshape(L * CD, L * CV).T.astype(jnp.bfloat16)
    bias_row = conv_b[:, None]

    ce2 = jnp.transpose(ce, (0, 2, 1, 3)).reshape(L * CD, T)
    return _tc_conv_assemble(ce2, word_raw, wbig, wids.reshape(T // 2048, 1, 2048),
                             bias_row, B, S, T, L, WD, CV)
